# baseline (speedup 1.0000x reference)
import functools

import numpy as np

import jax
import jax.numpy as jnp
from jax import lax
from jax.experimental import pallas as pl
from jax.experimental.pallas import tpu as pltpu
from jax.experimental.pallas import tpu_sc as plsc

_INJECT_PROB = 0.5
_NC = 2
_NS = 16
_NW = _NC * _NS


@functools.lru_cache(maxsize=None)
def _sampled_constants(batch: int, num_waveforms: int):
    with jax.ensure_compile_time_eval():
        return _sampled_constants_impl(batch, num_waveforms)


def _sampled_constants_impl(batch: int, num_waveforms: int):
    key = jax.random.key(42)
    k_mask, k_dec, k_psi, k_phi, k_idx = jax.random.split(key, 5)
    rvs = jax.random.uniform(k_mask, (batch,), dtype=jnp.float32)
    mask = np.asarray(rvs < _INJECT_PROB)
    n = int(mask.sum())
    u = jax.random.uniform(k_dec, (n,), minval=-1.0, maxval=1.0, dtype=jnp.float32)
    dec = np.asarray(jnp.arcsin(u))
    psi = np.asarray(jax.random.uniform(
        k_psi, (n,), minval=0.0, maxval=float(np.pi), dtype=jnp.float32))
    phi = np.asarray(jax.random.uniform(
        k_phi, (n,), minval=-float(np.pi), maxval=float(np.pi), dtype=jnp.float32))
    idx = np.asarray(jax.random.permutation(k_idx, num_waveforms)[:n]).astype(np.int32)
    return dec, psi, phi, idx, mask


@functools.lru_cache(maxsize=None)
def _build_gather2(n: int, b_pad: int, wave_len: int, batch: int):
    b_per_l = b_pad // _NS
    n_chunks = b_per_l // 8
    mesh = plsc.VectorSubcoreMesh(core_axis_name="c", subcore_axis_name="s",
                                  num_cores=_NC, num_subcores=_NS)

    @functools.partial(
        pl.kernel,
        mesh=mesh,
        out_type=[
            jax.ShapeDtypeStruct((n, wave_len), jnp.float32),
            jax.ShapeDtypeStruct((n, wave_len), jnp.float32),
            jax.ShapeDtypeStruct((n,), jnp.float32),
            jax.ShapeDtypeStruct((n,), jnp.float32),
            jax.ShapeDtypeStruct((n,), jnp.float32),
        ],
        scratch_types=[
            pltpu.VMEM((b_per_l,), jnp.int32),
            pltpu.VMEM((b_per_l, wave_len), jnp.float32),
            [pltpu.SemaphoreType.DMA] * n_chunks,
            pltpu.SemaphoreType.DMA,
        ],
    )
    def gather2(plus_hbm, cross_hbm, idx_hbm, dec_in, psi_in, phi_in,
                plus_out, cross_out, dec_out, psi_out, phi_out,
                idx_v, rows, gsems, sem_st):
        core = lax.axis_index("c")
        lane = lax.axis_index("s")
        base = lane * b_per_l
        pltpu.sync_copy(idx_hbm.at[pl.ds(base, b_per_l)], idx_v)

        full_lanes = n // b_per_l
        tail_rows = n - full_lanes * b_per_l

        def run(src, dst, consts):
            gs = [pltpu.async_copy(src.at[idx_v.at[pl.ds(k * 8, 8)]],
                                   rows.at[pl.ds(k * 8, 8)], gsems[k])
                  for k in range(n_chunks)]

            for lid, (c_src, c_dst) in enumerate(consts):
                @pl.when(lane == lid)
                def _(c_src=c_src, c_dst=c_dst):
                    pltpu.sync_copy(c_src, c_dst)

            @pl.when(lane < full_lanes)
            def _():
                sts = []
                for k in range(n_chunks):
                    gs[k].wait()
                    sts.append(pltpu.async_copy(
                        rows.at[pl.ds(k * 8, 8)],
                        dst.at[pl.ds(base + k * 8, 8)], sem_st))
                for st in sts:
                    st.wait()

            if tail_rows:
                @pl.when(lane == full_lanes)
                def _():
                    sts = []
                    for k in range(n_chunks):
                        cnt = max(0, min(8, tail_rows - k * 8))
                        gs[k].wait()
                        if cnt:
                            sts.append(pltpu.async_copy(
                                rows.at[pl.ds(k * 8, cnt)],
                                dst.at[pl.ds(base + k * 8, cnt)], sem_st))
                    for st in sts:
                        st.wait()

            @pl.when(lane > full_lanes)
            def _():
                for g in gs:
                    g.wait()

        @pl.when(core == 0)
        def _():
            run(plus_hbm, plus_out,
                [(dec_in, dec_out), (psi_in, psi_out)])

        @pl.when(core == 1)
        def _():
            run(cross_hbm, cross_out, [(phi_in, phi_out)])

    return gather2


def kernel(X, plus, cross):
    batch = X.shape[0]
    num_waveforms, wave_len = plus.shape
    dec, psi, phi, idx, mask = _sampled_constants(batch, num_waveforms)
    n = idx.shape[0]
    chunk = 8 * _NW
    b_pad = max(chunk, ((n + chunk - 1) // chunk) * chunk)
    idx_pad = np.zeros((b_pad,), np.int32)
    idx_pad[:n] = idx
    gather2 = _build_gather2(n, b_pad, wave_len, batch)
    plus_s, cross_s, dec_o, psi_o, phi_o = gather2(
        plus, cross, jnp.asarray(idx_pad),
        jnp.asarray(dec), jnp.asarray(psi), jnp.asarray(phi))
    return (dec_o, psi_o, phi_o, plus_s, cross_s, jnp.asarray(mask))


# speedup vs baseline: 1.0051x; 1.0051x over previous
import functools

import numpy as np

import jax
import jax.numpy as jnp
from jax import lax
from jax.experimental import pallas as pl
from jax.experimental.pallas import tpu as pltpu
from jax.experimental.pallas import tpu_sc as plsc

_INJECT_PROB = 0.5
_NC = 2
_NS = 16
_NW = _NC * _NS


@functools.lru_cache(maxsize=None)
def _sampled_constants(batch: int, num_waveforms: int):
    with jax.ensure_compile_time_eval():
        return _sampled_constants_impl(batch, num_waveforms)


def _sampled_constants_impl(batch: int, num_waveforms: int):
    key = jax.random.key(42)
    k_mask, k_dec, k_psi, k_phi, k_idx = jax.random.split(key, 5)
    rvs = jax.random.uniform(k_mask, (batch,), dtype=jnp.float32)
    mask = np.asarray(rvs < _INJECT_PROB)
    n = int(mask.sum())
    u = jax.random.uniform(k_dec, (n,), minval=-1.0, maxval=1.0, dtype=jnp.float32)
    dec = np.asarray(jnp.arcsin(u))
    psi = np.asarray(jax.random.uniform(
        k_psi, (n,), minval=0.0, maxval=float(np.pi), dtype=jnp.float32))
    phi = np.asarray(jax.random.uniform(
        k_phi, (n,), minval=-float(np.pi), maxval=float(np.pi), dtype=jnp.float32))
    idx = np.asarray(jax.random.permutation(k_idx, num_waveforms)[:n]).astype(np.int32)
    return dec, psi, phi, idx, mask


@functools.lru_cache(maxsize=None)
def _build_gather2(n: int, b_pad: int, wave_len: int, batch: int):
    b_per_l = b_pad // _NS
    n_chunks = b_per_l // 8
    mesh = plsc.VectorSubcoreMesh(core_axis_name="c", subcore_axis_name="s",
                                  num_cores=_NC, num_subcores=_NS)

    @functools.partial(
        pl.kernel,
        mesh=mesh,
        out_type=[
            jax.ShapeDtypeStruct((n, wave_len), jnp.float32),
            jax.ShapeDtypeStruct((n, wave_len), jnp.float32),
        ],
        scratch_types=[
            pltpu.VMEM((b_per_l,), jnp.int32),
            pltpu.VMEM((b_per_l, wave_len), jnp.float32),
            [pltpu.SemaphoreType.DMA] * n_chunks,
            [pltpu.SemaphoreType.DMA] * n_chunks,
            pltpu.SemaphoreType.DMA,
        ],
    )
    def gather2(plus_hbm, cross_hbm, idx_hbm,
                plus_out, cross_out,
                idx_v, rows, isems, gsems, sem_st):
        core = lax.axis_index("c")
        lane = lax.axis_index("s")
        base = lane * b_per_l
        ics = [pltpu.async_copy(idx_hbm.at[pl.ds(base + k * 8, 8)],
                                idx_v.at[pl.ds(k * 8, 8)], isems[k])
               for k in range(n_chunks)]

        full_lanes = n // b_per_l
        tail_rows = n - full_lanes * b_per_l

        def run(src, dst):
            gs = []
            for k in range(n_chunks):
                ics[k].wait()
                gs.append(pltpu.async_copy(src.at[idx_v.at[pl.ds(k * 8, 8)]],
                                           rows.at[pl.ds(k * 8, 8)], gsems[k]))

            @pl.when(lane < full_lanes)
            def _():
                sts = []
                for k in range(n_chunks):
                    gs[k].wait()
                    sts.append(pltpu.async_copy(
                        rows.at[pl.ds(k * 8, 8)],
                        dst.at[pl.ds(base + k * 8, 8)], sem_st))
                for st in sts:
                    st.wait()

            if tail_rows:
                @pl.when(lane == full_lanes)
                def _():
                    sts = []
                    for k in range(n_chunks):
                        cnt = max(0, min(8, tail_rows - k * 8))
                        gs[k].wait()
                        if cnt:
                            sts.append(pltpu.async_copy(
                                rows.at[pl.ds(k * 8, cnt)],
                                dst.at[pl.ds(base + k * 8, cnt)], sem_st))
                    for st in sts:
                        st.wait()

            @pl.when(lane > full_lanes)
            def _():
                for g in gs:
                    g.wait()

        @pl.when(core == 0)
        def _():
            run(plus_hbm, plus_out)

        @pl.when(core == 1)
        def _():
            run(cross_hbm, cross_out)

    return gather2


def kernel(X, plus, cross):
    batch = X.shape[0]
    num_waveforms, wave_len = plus.shape
    dec, psi, phi, idx, mask = _sampled_constants(batch, num_waveforms)
    n = idx.shape[0]
    chunk = 8 * _NW
    b_pad = max(chunk, ((n + chunk - 1) // chunk) * chunk)
    idx_pad = np.zeros((b_pad,), np.int32)
    idx_pad[:n] = idx
    gather2 = _build_gather2(n, b_pad, wave_len, batch)
    plus_s, cross_s = gather2(plus, cross, jnp.asarray(idx_pad))
    return (jnp.asarray(dec), jnp.asarray(psi), jnp.asarray(phi),
            plus_s, cross_s, jnp.asarray(mask))


# speedup vs baseline: 1.0148x; 1.0097x over previous
import functools

import numpy as np

import jax
import jax.numpy as jnp
from jax import lax
from jax.experimental import pallas as pl
from jax.experimental.pallas import tpu as pltpu
from jax.experimental.pallas import tpu_sc as plsc

_INJECT_PROB = 0.5
_NC = 2
_NS = 16
_NW = _NC * _NS


@functools.lru_cache(maxsize=None)
def _sampled_constants(batch: int, num_waveforms: int):
    with jax.ensure_compile_time_eval():
        return _sampled_constants_impl(batch, num_waveforms)


def _sampled_constants_impl(batch: int, num_waveforms: int):
    key = jax.random.key(42)
    k_mask, k_dec, k_psi, k_phi, k_idx = jax.random.split(key, 5)
    rvs = jax.random.uniform(k_mask, (batch,), dtype=jnp.float32)
    mask = np.asarray(rvs < _INJECT_PROB)
    n = int(mask.sum())
    u = jax.random.uniform(k_dec, (n,), minval=-1.0, maxval=1.0, dtype=jnp.float32)
    dec = np.asarray(jnp.arcsin(u))
    psi = np.asarray(jax.random.uniform(
        k_psi, (n,), minval=0.0, maxval=float(np.pi), dtype=jnp.float32))
    phi = np.asarray(jax.random.uniform(
        k_phi, (n,), minval=-float(np.pi), maxval=float(np.pi), dtype=jnp.float32))
    idx = np.asarray(jax.random.permutation(k_idx, num_waveforms)[:n]).astype(np.int32)
    return dec, psi, phi, idx, mask


@functools.lru_cache(maxsize=None)
def _build_gather2(n: int, b_pad: int, wave_len: int, batch: int):
    rows_per_w = b_pad // _NW
    mesh = plsc.VectorSubcoreMesh(core_axis_name="c", subcore_axis_name="s",
                                  num_cores=_NC, num_subcores=_NS)

    @functools.partial(
        pl.kernel,
        mesh=mesh,
        out_type=[
            jax.ShapeDtypeStruct((b_pad, wave_len), jnp.float32),
            jax.ShapeDtypeStruct((b_pad, wave_len), jnp.float32),
        ],
        scratch_types=[
            pltpu.VMEM((rows_per_w,), jnp.int32),
            pltpu.VMEM((rows_per_w, wave_len), jnp.float32),
            pltpu.VMEM((rows_per_w, wave_len), jnp.float32),
            pltpu.SemaphoreType.DMA,
            pltpu.SemaphoreType.DMA,
            pltpu.SemaphoreType.DMA,
        ],
    )
    def gather2(plus_hbm, cross_hbm, idx_hbm,
                plus_out, cross_out,
                idx_v, rows_p, rows_c, sem_p, sem_c, sem_st):
        core = lax.axis_index("c")
        lane = lax.axis_index("s")
        base = (core * _NS + lane) * rows_per_w
        pltpu.sync_copy(idx_hbm.at[pl.ds(base, rows_per_w)], idx_v)
        gp = pltpu.async_copy(plus_hbm.at[idx_v], rows_p, sem_p)
        gc = pltpu.async_copy(cross_hbm.at[idx_v], rows_c, sem_c)
        gp.wait()
        st_p = pltpu.async_copy(rows_p, plus_out.at[pl.ds(base, rows_per_w)],
                                sem_st)
        gc.wait()
        st_c = pltpu.async_copy(rows_c, cross_out.at[pl.ds(base, rows_per_w)],
                                sem_st)
        st_p.wait()
        st_c.wait()

    return gather2


def kernel(X, plus, cross):
    batch = X.shape[0]
    num_waveforms, wave_len = plus.shape
    dec, psi, phi, idx, mask = _sampled_constants(batch, num_waveforms)
    n = idx.shape[0]
    chunk = 8 * _NW
    b_pad = max(chunk, ((n + chunk - 1) // chunk) * chunk)
    idx_pad = np.zeros((b_pad,), np.int32)
    idx_pad[:n] = idx
    gather2 = _build_gather2(n, b_pad, wave_len, batch)
    plus_p, cross_p = gather2(plus, cross, jnp.asarray(idx_pad))
    return (jnp.asarray(dec), jnp.asarray(psi), jnp.asarray(phi),
            plus_p[:n], cross_p[:n], jnp.asarray(mask))
